# baseline (speedup 1.0000x reference)
import functools

import jax
import jax.numpy as jnp
from jax import lax
from jax.experimental import pallas as pl
from jax.experimental.pallas import tpu as pltpu
from jax.experimental.pallas import tpu_sc as plsc

F = 26
V = 100000
D = 32
NC = 2
NS = 16
L = 16
NW = NC * NS

SLICE = V // NW
WIN = 3136
MP = 2048
UN = 8
OB = 256


def _rsqrt_nr(x):
    i = plsc.bitcast(x, jnp.int32)
    i = jnp.int32(0x5F3759DF) - (i >> 1)
    y = plsc.bitcast(i, jnp.float32)
    for _ in range(3):
        y = y * (1.5 - 0.5 * x * y * y)
    return y


@functools.lru_cache(maxsize=None)
def _build(B):
    assert B % MP == 0 and V % NW == 0
    pieces = B // MP

    mesh = plsc.VectorSubcoreMesh(
        core_axis_name="c", subcore_axis_name="s",
        num_cores=NC, num_subcores=NS)

    @functools.partial(
        pl.kernel,
        out_type=jax.ShapeDtypeStruct((B * F, D), jnp.float32),
        mesh=mesh,
        compiler_params=pltpu.CompilerParams(
            needs_layout_passes=False, use_tc_tiling_on_sc=False),
        scratch_types=[
            pltpu.VMEM((D, WIN), jnp.float32),
            pltpu.VMEM((MP,), jnp.int32),
            pltpu.VMEM((MP,), jnp.int32),
            pltpu.VMEM((B,), jnp.int32),
            pltpu.VMEM((OB, D), jnp.float32),
            pltpu.VMEM((OB // 128, 128), jnp.int32),
            pltpu.SemaphoreType.DMA,
            pltpu.SemaphoreType.DMA,
            pltpu.SemaphoreType.DMA,
        ],
    )
    def emb_kernel(metas_hbm, tt_hbm, out_hbm,
                   slab, mrow0, mrow1, items, orows, oidx,
                   ssem, osem, msem):
        wid = lax.axis_index("s") * NC + lax.axis_index("c")
        v0 = wid * SLICE
        win0 = jnp.minimum((v0 // 8) * 8, V - WIN)

        def field(f, _):
            stage = pltpu.make_async_copy(
                tt_hbm.at[pl.ds(f * D, D), pl.ds(win0, WIN)], slab, ssem)
            stage.start()

            mrows = [mrow0, mrow1]
            mcopies = [
                pltpu.make_async_copy(
                    metas_hbm.at[pl.ds(f * B + p * MP, MP)],
                    mrows[p % 2], msem)
                for p in range(pieces)
            ]
            mcopies[0].start()
            cnt = 0
            for p in range(pieces):
                if p + 1 < pieces:
                    mcopies[p + 1].start()
                mcopies[p].wait()
                mrow = mrows[p % 2]

                def scan(g, cnt, p=p, mrow=mrow):
                    for u in range(UN):
                        off = g * L * UN + u * L
                        v16 = mrow[pl.ds(off, L)]
                        rel = (v16 - v0).astype(jnp.uint32)
                        mask = rel < jnp.uint32(SLICE)
                        b16 = lax.iota(jnp.int32, L) + (p * MP + off)
                        code = (v16 - win0) + (b16 << 12)
                        plsc.store_compressed(
                            items.at[pl.ds(cnt, L)], code, mask=mask)
                        cnt = cnt + \
                            plsc.all_reduce_population_count(mask)[0]
                    return cnt

                cnt = lax.fori_loop(0, MP // (L * UN), scan, cnt)
            stage.wait()

            def run_batches(items, cnt):
              def batch(q, _):
                base = q * OB
                last = jnp.maximum(cnt - L, 0)
                for g2 in range(OB // L):
                    gb = jnp.minimum(base + g2 * L, last)
                    lanepos = gb + lax.iota(jnp.int32, L)
                    valid = lanepos < cnt
                    code = items[pl.ds(gb, L)]
                    code = jnp.where(valid, code, code[0])
                    vloc = jnp.minimum(code & 4095, WIN - 1)
                    bb = code >> 12
                    vals = []
                    acc = jnp.zeros((L,), jnp.float32)
                    for d in range(D):
                        dv = jnp.full((L,), d, jnp.int32)
                        x = plsc.load_gather(slab, [dv, vloc])
                        vals.append(x)
                        acc = acc + x * x
                    inv = _rsqrt_nr(acc)
                    inv = jnp.where(acc <= 1e-16, 1.0, inv)
                    rr = jnp.full((L,), g2 * L, jnp.int32) + \
                        lax.iota(jnp.int32, L)
                    for d in range(D):
                        dv = jnp.full((L,), d, jnp.int32)
                        plsc.store_scatter(orows, [rr, dv], vals[d] * inv)
                    oidx[g2 // 8, pl.ds((g2 % 8) * L, L)] = bb * F + f
                copies = [
                    pltpu.make_async_copy(
                        orows.at[pl.ds(j * 128, 128)],
                        out_hbm.at[oidx.at[j]],
                        osem)
                    for j in range(OB // 128)
                ]
                for c in copies:
                    c.start()
                for c in copies:
                    c.wait()
                return 0

              nbatches = (cnt + OB - 1) // OB
              lax.fori_loop(0, nbatches, batch, 0)

            run_batches(items, cnt)
            return 0

        lax.fori_loop(0, F, field, 0)

    return emb_kernel


def kernel(metas, tables):
    B = metas.shape[0]
    metas_fm = metas.T.reshape(-1)
    tt = jax.lax.optimization_barrier(
        tables.transpose(0, 2, 1).reshape(F * D, V))
    out = _build(B)(metas_fm, tt)
    return out.reshape(B, F * D)


# speedup vs baseline: 1.0762x; 1.0762x over previous
import functools

import jax
import jax.numpy as jnp
from jax import lax
from jax.experimental import pallas as pl
from jax.experimental.pallas import tpu as pltpu
from jax.experimental.pallas import tpu_sc as plsc

F = 26
V = 100000
D = 32
NC = 2
NS = 16
L = 16
NW = NC * NS

SLICE = V // NW
WIN = 3136
MP = 2048
UN = 8
OB = 128


def _rsqrt_nr(x):
    i = plsc.bitcast(x, jnp.int32)
    i = jnp.int32(0x5F3759DF) - (i >> 1)
    y = plsc.bitcast(i, jnp.float32)
    for _ in range(3):
        y = y * (1.5 - 0.5 * x * y * y)
    return y


@functools.lru_cache(maxsize=None)
def _build(B):
    assert B % MP == 0 and V % NW == 0
    pieces = B // MP

    mesh = plsc.VectorSubcoreMesh(
        core_axis_name="c", subcore_axis_name="s",
        num_cores=NC, num_subcores=NS)

    @functools.partial(
        pl.kernel,
        out_type=jax.ShapeDtypeStruct((B * F, D), jnp.float32),
        mesh=mesh,
        compiler_params=pltpu.CompilerParams(
            needs_layout_passes=False, use_tc_tiling_on_sc=False),
        scratch_types=[
            pltpu.VMEM((D, WIN), jnp.float32),
            pltpu.VMEM((MP,), jnp.int32),
            pltpu.VMEM((MP,), jnp.int32),
            pltpu.VMEM((B,), jnp.int32),
            pltpu.VMEM((OB, D), jnp.float32),
            pltpu.VMEM((OB // 128, 128), jnp.int32),
            pltpu.SemaphoreType.DMA,
            pltpu.SemaphoreType.DMA,
            pltpu.SemaphoreType.DMA,
        ],
    )
    def emb_kernel(metas_hbm, tt_hbm, out_hbm,
                   slab, mrow0, mrow1, items, orows, oidx,
                   ssem, osem, msem):
        wid = lax.axis_index("s") * NC + lax.axis_index("c")
        v0 = wid * SLICE
        win0 = jnp.minimum((v0 // 8) * 8, V - WIN)

        def field(f, _):
            stage = pltpu.make_async_copy(
                tt_hbm.at[pl.ds(f * D, D), pl.ds(win0, WIN)], slab, ssem)
            stage.start()

            mrows = [mrow0, mrow1]
            mcopies = [
                pltpu.make_async_copy(
                    metas_hbm.at[pl.ds(f * B + p * MP, MP)],
                    mrows[p % 2], msem)
                for p in range(pieces)
            ]
            mcopies[0].start()
            cnt = 0
            for p in range(pieces):
                if p + 1 < pieces:
                    mcopies[p + 1].start()
                mcopies[p].wait()
                mrow = mrows[p % 2]

                def scan(g, cnt, p=p, mrow=mrow):
                    for u in range(UN):
                        off = g * L * UN + u * L
                        v16 = mrow[pl.ds(off, L)]
                        rel = (v16 - v0).astype(jnp.uint32)
                        mask = rel < jnp.uint32(SLICE)
                        b16 = lax.iota(jnp.int32, L) + (p * MP + off)
                        code = (v16 - win0) + (b16 << 12)
                        plsc.store_compressed(
                            items.at[pl.ds(cnt, L)], code, mask=mask)
                        cnt = cnt + \
                            plsc.all_reduce_population_count(mask)[0]
                    return cnt

                cnt = lax.fori_loop(0, MP // (L * UN), scan, cnt)
            stage.wait()

            def run_batches(items, cnt):
              def batch(q, _):
                base = q * OB
                last = jnp.maximum(cnt - L, 0)
                for g2 in range(OB // L):
                    gb = jnp.minimum(base + g2 * L, last)
                    lanepos = gb + lax.iota(jnp.int32, L)
                    valid = lanepos < cnt
                    code = items[pl.ds(gb, L)]
                    code = jnp.where(valid, code, code[0])
                    vloc = jnp.minimum(code & 4095, WIN - 1)
                    bb = code >> 12
                    vals = []
                    acc = jnp.zeros((L,), jnp.float32)
                    for d in range(D):
                        dv = jnp.full((L,), d, jnp.int32)
                        x = plsc.load_gather(slab, [dv, vloc])
                        vals.append(x)
                        acc = acc + x * x
                    inv = _rsqrt_nr(acc)
                    inv = jnp.where(acc <= 1e-16, 1.0, inv)
                    rr = jnp.full((L,), g2 * L, jnp.int32) + \
                        lax.iota(jnp.int32, L)
                    for d in range(D):
                        dv = jnp.full((L,), d, jnp.int32)
                        plsc.store_scatter(orows, [rr, dv], vals[d] * inv)
                    oidx[g2 // 8, pl.ds((g2 % 8) * L, L)] = bb * F + f
                copies = [
                    pltpu.make_async_copy(
                        orows.at[pl.ds(j * 128, 128)],
                        out_hbm.at[oidx.at[j]],
                        osem)
                    for j in range(OB // 128)
                ]
                for c in copies:
                    c.start()
                for c in copies:
                    c.wait()
                return 0

              nbatches = (cnt + OB - 1) // OB
              lax.fori_loop(0, nbatches, batch, 0)

            run_batches(items, cnt)
            return 0

        lax.fori_loop(0, F, field, 0)

    return emb_kernel


def kernel(metas, tables):
    B = metas.shape[0]
    metas_fm = metas.T.reshape(-1)
    tt = jax.lax.optimization_barrier(
        tables.transpose(0, 2, 1).reshape(F * D, V))
    out = _build(B)(metas_fm, tt)
    return out.reshape(B, F * D)
